# baseline (speedup 1.0000x reference)
import functools

import jax
import jax.numpy as jnp
from jax.experimental import pallas as pl
from jax.experimental.pallas import tpu as pltpu


def _fused(x0in_ref, inc_ref, wt_ref, b_ref,
           xlin_ref, x1_ref, out_ref,
           xlinbf_ref, nacc_ref, *, e_total):
    i = pl.program_id(0)

    @pl.when(i == 0)
    def _():
        xlin = (
            jnp.dot(x0in_ref[...], wt_ref[...], preferred_element_type=jnp.float32)
            + b_ref[...]
        )
        xlin_ref[...] = xlin
        xlinbf_ref[...] = xlin.T.astype(jnp.bfloat16)
        out_ref[...] = jnp.zeros_like(out_ref)
        nacc_ref[...] = jnp.zeros_like(nacc_ref)

    col = (jax.lax.broadcasted_iota(jnp.int32, (1, inc_ref.shape[1]), 1)
           + i * inc_ref.shape[1])
    blkbf = jnp.where(col < e_total, inc_ref[...], 0.0).astype(jnp.bfloat16)

    x1t = jnp.dot(xlinbf_ref[...], blkbf, preferred_element_type=jnp.float32)
    x1 = x1t.T
    x1_ref[...] = x1
    out_ref[...] += jnp.dot(blkbf, x1.astype(jnp.bfloat16),
                            preferred_element_type=jnp.float32)
    nacc_ref[...] += jnp.sum(blkbf, axis=1, keepdims=True,
                             dtype=jnp.float32)

    @pl.when(i == pl.num_programs(0) - 1)
    def _():
        out_ref[...] = xlin_ref[...] + out_ref[...] / nacc_ref[...]


def kernel(x_0, incidence_1, W, b):
    n, c_in = x_0.shape
    e = incidence_1.shape[1]
    c_hid = W.shape[0]
    wt = W.T
    b2 = b.reshape(1, c_hid)

    be = min(384, e)
    xlin, x_1, x0_out = pl.pallas_call(
        functools.partial(_fused, e_total=e),
        grid=(pl.cdiv(e, be),),
        in_specs=[
            pl.BlockSpec((n, c_in), lambda i: (0, 0)),
            pl.BlockSpec((n, be), lambda i: (0, i)),
            pl.BlockSpec((c_in, c_hid), lambda i: (0, 0)),
            pl.BlockSpec((1, c_hid), lambda i: (0, 0)),
        ],
        out_specs=[
            pl.BlockSpec((n, c_hid), lambda i: (0, 0)),
            pl.BlockSpec((be, c_hid), lambda i: (i, 0)),
            pl.BlockSpec((n, c_hid), lambda i: (0, 0)),
        ],
        out_shape=[
            jax.ShapeDtypeStruct((n, c_hid), jnp.float32),
            jax.ShapeDtypeStruct((e, c_hid), jnp.float32),
            jax.ShapeDtypeStruct((n, c_hid), jnp.float32),
        ],
        scratch_shapes=[
            pltpu.VMEM((c_hid, n), jnp.bfloat16),
            pltpu.VMEM((n, 1), jnp.float32),
        ],
        compiler_params=pltpu.CompilerParams(
            vmem_limit_bytes=63 * 1024 * 1024),
    )(x_0, incidence_1, wt, b2)

    return (x0_out, x_1)


# speedup vs baseline: 1.0563x; 1.0563x over previous
import functools

import jax
import jax.numpy as jnp
from jax.experimental import pallas as pl
from jax.experimental.pallas import tpu as pltpu


def _fused(x0in_ref, inc_ref, wt_ref, b_ref,
           xlin_ref, x1_ref, out_ref,
           xlinbf_ref, nacc_ref, *, e_total):
    i = pl.program_id(0)

    @pl.when(i == 0)
    def _():
        xlin = (
            jnp.dot(x0in_ref[...], wt_ref[...], preferred_element_type=jnp.float32)
            + b_ref[...]
        )
        xlin_ref[...] = xlin
        xlinbf_ref[...] = xlin.T.astype(jnp.bfloat16)
        out_ref[...] = jnp.zeros_like(out_ref)
        nacc_ref[...] = jnp.zeros_like(nacc_ref)

    col = (jax.lax.broadcasted_iota(jnp.int32, (1, inc_ref.shape[1]), 1)
           + i * inc_ref.shape[1])
    blkm = jnp.where(col < e_total, inc_ref[...], 0.0)
    blkbf = blkm.astype(jnp.bfloat16)

    x1t = jnp.dot(xlinbf_ref[...], blkbf, preferred_element_type=jnp.float32)
    x1 = x1t.T
    x1_ref[...] = x1
    out_ref[...] += jnp.dot(blkbf, x1.astype(jnp.bfloat16),
                            preferred_element_type=jnp.float32)
    nacc_ref[...] += jnp.sum(blkm, axis=1, keepdims=True)

    @pl.when(i == pl.num_programs(0) - 1)
    def _():
        out_ref[...] = xlin_ref[...] + out_ref[...] / nacc_ref[...]


def kernel(x_0, incidence_1, W, b):
    n, c_in = x_0.shape
    e = incidence_1.shape[1]
    c_hid = W.shape[0]
    wt = W.T
    b2 = b.reshape(1, c_hid)

    be = min(384, e)
    xlin, x_1, x0_out = pl.pallas_call(
        functools.partial(_fused, e_total=e),
        grid=(pl.cdiv(e, be),),
        in_specs=[
            pl.BlockSpec((n, c_in), lambda i: (0, 0)),
            pl.BlockSpec((n, be), lambda i: (0, i)),
            pl.BlockSpec((c_in, c_hid), lambda i: (0, 0)),
            pl.BlockSpec((1, c_hid), lambda i: (0, 0)),
        ],
        out_specs=[
            pl.BlockSpec((n, c_hid), lambda i: (0, 0)),
            pl.BlockSpec((be, c_hid), lambda i: (i, 0)),
            pl.BlockSpec((n, c_hid), lambda i: (0, 0)),
        ],
        out_shape=[
            jax.ShapeDtypeStruct((n, c_hid), jnp.float32),
            jax.ShapeDtypeStruct((e, c_hid), jnp.float32),
            jax.ShapeDtypeStruct((n, c_hid), jnp.float32),
        ],
        scratch_shapes=[
            pltpu.VMEM((c_hid, n), jnp.bfloat16),
            pltpu.VMEM((n, 1), jnp.float32),
        ],
        compiler_params=pltpu.CompilerParams(
            vmem_limit_bytes=63 * 1024 * 1024),
    )(x_0, incidence_1, wt, b2)

    return (x0_out, x_1)
